# baseline (speedup 1.0000x reference)
import functools

import jax
import jax.numpy as jnp
from jax import lax
from jax.experimental import pallas as pl
from jax.experimental.pallas import tpu as pltpu
from jax.experimental.pallas import tpu_sc as plsc

VOCAB = 1_000_000
D = 64
NC = 2
NS = 16
NW = NC * NS
NB = 4096
NT = 200
ROW = 128
NUNITS = (NT // 8) * (NB // ROW) * 8
UPW = NUNITS // NW
HALF = UPW // 2


def _embed_body(idx_hbm, w_hbm, out_hbm, idx_v, g_a, g_b, p_a, p_b,
                gsem_a, gsem_b, wsem_a, wsem_b):
    wid = lax.axis_index("s") * NC + lax.axis_index("c")
    vbase = pl.multiple_of(wid * UPW, 8)
    pltpu.sync_copy(idx_hbm.at[pl.ds(vbase, UPW)], idx_v)

    lanes = lax.iota(jnp.int32, 16)

    def fire_g(u, gbuf, gsem):
        pltpu.async_copy(w_hbm.at[idx_v.at[u]], gbuf, gsem)

    def drain_g(gbuf, gsem):
        pltpu.make_async_copy(w_hbm.at[pl.ds(0, ROW)], gbuf, gsem).wait()

    def transpose(gbuf, pbuf):
        for fr in range(8):
            def inner(bg, _):
                rows16 = bg * 16 + lanes
                for fs in range(8):
                    vals = plsc.load_gather(
                        gbuf, [rows16, jnp.full((16,), fr * 8 + fs, jnp.int32)])
                    pbuf.at[fr].at[fs][pl.ds(bg * 16, 16)] = vals
                return 0
            lax.fori_loop(0, 8, inner, 0)

    def fire_w(u, pbuf, wsem):
        v = vbase + u
        t = (v // 256) * 8 + lax.rem(v, 8)
        bb = lax.rem(v // 8, 32)
        for fr in range(8):
            pltpu.async_copy(pbuf.at[fr], out_hbm.at[t, fr, bb], wsem)

    def drain_w(pbuf, wsem):
        for fr in range(8):
            pltpu.make_async_copy(pbuf.at[fr], out_hbm.at[0, fr, 0],
                                  wsem).wait()

    def unit(u, k2, gbuf, gsem, pbuf, wsem):
        drain_g(gbuf, gsem)

        @pl.when(k2 > 0)
        def _():
            drain_w(pbuf, wsem)

        transpose(gbuf, pbuf)

        @pl.when(k2 < HALF - 1)
        def _():
            fire_g(u + 2, gbuf, gsem)

        fire_w(u, pbuf, wsem)

    fire_g(0, g_a, gsem_a)
    fire_g(1, g_b, gsem_b)

    def pair(k2, _):
        unit(2 * k2, k2, g_a, gsem_a, p_a, wsem_a)
        unit(2 * k2 + 1, k2, g_b, gsem_b, p_b, wsem_b)
        return 0

    lax.fori_loop(0, HALF, pair, 0)
    drain_w(p_a, wsem_a)
    drain_w(p_b, wsem_b)


_embed = functools.partial(
    pl.kernel,
    mesh=plsc.VectorSubcoreMesh(core_axis_name="c", subcore_axis_name="s"),
    out_type=jax.ShapeDtypeStruct((NT, 8, NB // ROW, 8, ROW), jnp.float32),
    scratch_types=[
        pltpu.VMEM((UPW, ROW), jnp.int32),
        pltpu.VMEM((ROW, D), jnp.float32),
        pltpu.VMEM((ROW, D), jnp.float32),
        pltpu.VMEM((8, 8, ROW), jnp.float32),
        pltpu.VMEM((8, 8, ROW), jnp.float32),
        pltpu.SemaphoreType.DMA,
        pltpu.SemaphoreType.DMA,
        pltpu.SemaphoreType.DMA,
        pltpu.SemaphoreType.DMA,
    ],
    compiler_params=pltpu.CompilerParams(use_tc_tiling_on_sc=False,
                                         needs_layout_passes=False),
)(_embed_body)


def kernel(idx, weight):
    idx5 = (idx.T.astype(jnp.int32)
            .reshape(NT // 8, 8, NB // ROW, ROW)
            .transpose(0, 2, 1, 3)
            .reshape(NUNITS, ROW))
    out5 = _embed(idx5, weight)
    return (out5.transpose(2, 4, 0, 1, 3)
            .reshape(NB, NT, D))


# speedup vs baseline: 1.3825x; 1.3825x over previous
import functools

import jax
import jax.numpy as jnp
from jax import lax
from jax.experimental import pallas as pl
from jax.experimental.pallas import tpu as pltpu
from jax.experimental.pallas import tpu_sc as plsc

VOCAB = 1_000_000
D = 64
NC = 2
NS = 16
NW = NC * NS
NB = 4096
NT = 200
ROW = 128
NUNITS = (NT // 8) * (NB // ROW) * 8
UPW = NUNITS // NW
HALF = UPW // 2


def _embed_body(idx_hbm, w_hbm, out_hbm, idx_v, g_a, g_b, p_a, p_b,
                gsem_a, gsem_b, wsem_a, wsem_b):
    wid = lax.axis_index("s") * NC + lax.axis_index("c")
    vbase = pl.multiple_of(wid * UPW, 8)
    pltpu.sync_copy(idx_hbm.at[pl.ds(vbase, UPW)], idx_v)

    lanes = lax.iota(jnp.int32, 16)

    def fire_g(u, gbuf, gsem):
        pltpu.async_copy(w_hbm.at[idx_v.at[u]], gbuf, gsem)

    def drain_g(gbuf, gsem):
        pltpu.make_async_copy(w_hbm.at[pl.ds(0, ROW)], gbuf, gsem).wait()

    def transpose(gbuf, pbuf):
        for fr in range(8):
            def inner(bg, _):
                rows16 = bg * 16 + lanes
                vals = [
                    plsc.load_gather(
                        gbuf, [rows16, jnp.full((16,), fr * 8 + fs, jnp.int32)])
                    for fs in range(8)
                ]
                for fs in range(8):
                    pbuf.at[fr].at[fs][pl.ds(bg * 16, 16)] = vals[fs]
                return 0
            lax.fori_loop(0, 8, inner, 0)

    def fire_w(u, pbuf, wsem):
        v = vbase + u
        t = (v // 256) * 8 + lax.rem(v, 8)
        bb = lax.rem(v // 8, 32)
        for fr in range(8):
            pltpu.async_copy(pbuf.at[fr], out_hbm.at[t, fr, bb], wsem)

    def drain_w(pbuf, wsem):
        for fr in range(8):
            pltpu.make_async_copy(pbuf.at[fr], out_hbm.at[0, fr, 0],
                                  wsem).wait()

    def unit(u, k2, gbuf, gsem, pbuf, wsem):
        drain_g(gbuf, gsem)

        @pl.when(k2 > 0)
        def _():
            drain_w(pbuf, wsem)

        transpose(gbuf, pbuf)

        @pl.when(k2 < HALF - 1)
        def _():
            fire_g(u + 2, gbuf, gsem)

        fire_w(u, pbuf, wsem)

    fire_g(0, g_a, gsem_a)
    fire_g(1, g_b, gsem_b)

    def pair(k2, _):
        unit(2 * k2, k2, g_a, gsem_a, p_a, wsem_a)
        unit(2 * k2 + 1, k2, g_b, gsem_b, p_b, wsem_b)
        return 0

    lax.fori_loop(0, HALF, pair, 0)
    drain_w(p_a, wsem_a)
    drain_w(p_b, wsem_b)


_embed = functools.partial(
    pl.kernel,
    mesh=plsc.VectorSubcoreMesh(core_axis_name="c", subcore_axis_name="s"),
    out_type=jax.ShapeDtypeStruct((NT, 8, NB // ROW, 8, ROW), jnp.float32),
    scratch_types=[
        pltpu.VMEM((UPW, ROW), jnp.int32),
        pltpu.VMEM((ROW, D), jnp.float32),
        pltpu.VMEM((ROW, D), jnp.float32),
        pltpu.VMEM((8, 8, ROW), jnp.float32),
        pltpu.VMEM((8, 8, ROW), jnp.float32),
        pltpu.SemaphoreType.DMA,
        pltpu.SemaphoreType.DMA,
        pltpu.SemaphoreType.DMA,
        pltpu.SemaphoreType.DMA,
    ],
    compiler_params=pltpu.CompilerParams(use_tc_tiling_on_sc=False,
                                         needs_layout_passes=False),
)(_embed_body)


def kernel(idx, weight):
    idx5 = (idx.T.astype(jnp.int32)
            .reshape(NT // 8, 8, NB // ROW, ROW)
            .transpose(0, 2, 1, 3)
            .reshape(NUNITS, ROW))
    out5 = _embed(idx5, weight)
    return (out5.transpose(2, 4, 0, 1, 3)
            .reshape(NB, NT, D))


# speedup vs baseline: 1.3825x; 1.0000x over previous
import functools

import jax
import jax.numpy as jnp
from jax import lax
from jax.experimental import pallas as pl
from jax.experimental.pallas import tpu as pltpu
from jax.experimental.pallas import tpu_sc as plsc

VOCAB = 1_000_000
D = 64
NC = 2
NS = 16
NW = NC * NS
NB = 4096
NT = 200
ROW = 128
NUNITS = (NT // 8) * (NB // ROW) * 8
UPW = NUNITS // NW
HALF = UPW // 2


GRP = 4
NGRP = UPW // GRP
NPAIR = NGRP // 2


def _embed_body(idx_hbm, w_hbm, out_hbm, idx_v, g_bufs, p_a, p_b,
                gsem_a, gsem_b, wsem_a, wsem_b):
    wid = lax.axis_index("s") * NC + lax.axis_index("c")
    vbase = pl.multiple_of(wid * UPW, 8)
    pltpu.sync_copy(idx_hbm.at[pl.ds(vbase, UPW)], idx_v)

    lanes = lax.iota(jnp.int32, 16)
    gsem = (gsem_a, gsem_b)
    wsem = (wsem_a, wsem_b)
    pbuf = (p_a, p_b)

    def fire_group(g, s):
        for j in range(GRP):
            pltpu.async_copy(w_hbm.at[idx_v.at[g * GRP + j]],
                             g_bufs.at[s * GRP + j], gsem[s])

    def drain_group(s):
        for j in range(GRP):
            pltpu.make_async_copy(w_hbm.at[pl.ds(0, ROW)], g_bufs.at[0],
                                  gsem[s]).wait()

    def transpose(gbuf, pb):
        for fr in range(8):
            def inner(bg, _):
                rows16 = bg * 16 + lanes
                vals = [
                    plsc.load_gather(
                        gbuf, [rows16, jnp.full((16,), fr * 8 + fs, jnp.int32)])
                    for fs in range(8)
                ]
                for fs in range(8):
                    pb.at[fr].at[fs][pl.ds(bg * 16, 16)] = vals[fs]
                return 0
            lax.fori_loop(0, 8, inner, 0)

    def fire_w(u, p):
        v = vbase + u
        t = (v // 256) * 8 + lax.rem(v, 8)
        bb = lax.rem(v // 8, 32)
        for fr in range(8):
            pltpu.async_copy(pbuf[p].at[fr], out_hbm.at[t, fr, bb], wsem[p])

    def drain_w(p):
        for fr in range(8):
            pltpu.make_async_copy(pbuf[p].at[fr], out_hbm.at[0, fr, 0],
                                  wsem[p]).wait()

    def process_unit(u, j, s, k2, guard_first):
        p = j % 2
        if guard_first and j < 2:
            @pl.when(k2 > 0)
            def _():
                drain_w(p)
        else:
            drain_w(p)
        transpose(g_bufs.at[s * GRP + j], pbuf[p])
        fire_w(u, p)

    fire_group(0, 0)

    def pair(k2, _):
        g0 = 2 * k2
        drain_group(0)
        fire_group(g0 + 1, 1)
        for j in range(GRP):
            process_unit(g0 * GRP + j, j, 0, k2, True)
        drain_group(1)

        @pl.when(k2 < NPAIR - 1)
        def _():
            fire_group(g0 + 2, 0)

        for j in range(GRP):
            process_unit((g0 + 1) * GRP + j, j, 1, k2, False)
        return 0

    lax.fori_loop(0, NPAIR, pair, 0)
    drain_w(0)
    drain_w(1)


_embed = functools.partial(
    pl.kernel,
    mesh=plsc.VectorSubcoreMesh(core_axis_name="c", subcore_axis_name="s"),
    out_type=jax.ShapeDtypeStruct((NT, 8, NB // ROW, 8, ROW), jnp.float32),
    scratch_types=[
        pltpu.VMEM((UPW, ROW), jnp.int32),
        pltpu.VMEM((2 * GRP, ROW, D), jnp.float32),
        pltpu.VMEM((8, 8, ROW), jnp.float32),
        pltpu.VMEM((8, 8, ROW), jnp.float32),
        pltpu.SemaphoreType.DMA,
        pltpu.SemaphoreType.DMA,
        pltpu.SemaphoreType.DMA,
        pltpu.SemaphoreType.DMA,
    ],
    compiler_params=pltpu.CompilerParams(use_tc_tiling_on_sc=False,
                                         needs_layout_passes=False),
)(_embed_body)


def kernel(idx, weight):
    idx5 = (idx.T.astype(jnp.int32)
            .reshape(NT // 8, 8, NB // ROW, ROW)
            .transpose(0, 2, 1, 3)
            .reshape(NUNITS, ROW))
    out5 = _embed(idx5, weight)
    return (out5.transpose(2, 4, 0, 1, 3)
            .reshape(NB, NT, D))


# speedup vs baseline: 2.3001x; 1.6637x over previous
import functools

import jax
import jax.numpy as jnp
from jax import lax
from jax.experimental import pallas as pl
from jax.experimental.pallas import tpu as pltpu
from jax.experimental.pallas import tpu_sc as plsc

VOCAB = 1_000_000
D = 64
NC = 2
NS = 16
NW = NC * NS
NB = 4096
NT = 200
ROW = 128
NUNITS = (NT // 8) * (NB // ROW) * 8
UPW = NUNITS // NW
HALF = UPW // 2


GRP = 4
NGRP = UPW // GRP
NPAIR = NGRP // 2


def _embed_body(idx_hbm, w_hbm, out_hbm, idx_v, g_bufs, p_a, p_b,
                gsem_a, gsem_b, wsem_a, wsem_b):
    wid = lax.axis_index("s") * NC + lax.axis_index("c")
    vbase = pl.multiple_of(wid * UPW, 8)
    pltpu.sync_copy(idx_hbm.at[pl.ds(vbase, UPW)], idx_v)

    lanes = lax.iota(jnp.int32, 16)
    gsem = (gsem_a, gsem_b)
    wsem = (wsem_a, wsem_b)
    pbuf = (p_a, p_b)

    def fire_group(g, s):
        for j in range(GRP):
            pltpu.async_copy(w_hbm.at[idx_v.at[g * GRP + j]],
                             g_bufs.at[s * GRP + j], gsem[s])

    def drain_group(s):
        for j in range(GRP):
            pltpu.make_async_copy(w_hbm.at[pl.ds(0, ROW)], g_bufs.at[0],
                                  gsem[s]).wait()

    rots = [lax.rem(lanes + k, 16) for k in range(16)]

    def transpose(gbuf, pb):
        def inner(bb2, _):
            bl_vec = bb2 * 16 + lanes
            for d0 in (0, 16, 32, 48):
                for half in (0, 8):
                    dvs = [jnp.bitwise_or(rots[half + k], d0)
                           for k in range(8)]
                    vals = [plsc.load_gather(gbuf, [bl_vec, dv])
                            for dv in dvs]
                    for dv, v in zip(dvs, vals):
                        plsc.store_scatter(pb, [dv, bl_vec], v)
            return 0
        lax.fori_loop(0, 8, inner, 0)

    def fire_w(u, p):
        v = vbase + u
        t = (v // 256) * 8 + lax.rem(v, 8)
        bb = lax.rem(v // 8, 32)
        for fr in range(8):
            pltpu.async_copy(pbuf[p].at[pl.ds(fr * 8, 8)],
                             out_hbm.at[t, fr, bb], wsem[p])

    def drain_w(p):
        for fr in range(8):
            pltpu.make_async_copy(pbuf[p].at[pl.ds(fr * 8, 8)],
                                  out_hbm.at[0, fr, 0], wsem[p]).wait()

    def process_unit(u, j, s, k2, guard_first):
        p = j % 2
        if guard_first and j < 2:
            @pl.when(k2 > 0)
            def _():
                drain_w(p)
        else:
            drain_w(p)
        transpose(g_bufs.at[s * GRP + j], pbuf[p])
        fire_w(u, p)

    fire_group(0, 0)

    def pair(k2, _):
        g0 = 2 * k2
        drain_group(0)
        fire_group(g0 + 1, 1)
        for j in range(GRP):
            process_unit(g0 * GRP + j, j, 0, k2, True)
        drain_group(1)

        @pl.when(k2 < NPAIR - 1)
        def _():
            fire_group(g0 + 2, 0)

        for j in range(GRP):
            process_unit((g0 + 1) * GRP + j, j, 1, k2, False)
        return 0

    lax.fori_loop(0, NPAIR, pair, 0)
    drain_w(0)
    drain_w(1)


_embed = functools.partial(
    pl.kernel,
    mesh=plsc.VectorSubcoreMesh(core_axis_name="c", subcore_axis_name="s"),
    out_type=jax.ShapeDtypeStruct((NT, 8, NB // ROW, 8, ROW), jnp.float32),
    scratch_types=[
        pltpu.VMEM((UPW, ROW), jnp.int32),
        pltpu.VMEM((2 * GRP, ROW, D), jnp.float32),
        pltpu.VMEM((D, ROW), jnp.float32),
        pltpu.VMEM((D, ROW), jnp.float32),
        pltpu.SemaphoreType.DMA,
        pltpu.SemaphoreType.DMA,
        pltpu.SemaphoreType.DMA,
        pltpu.SemaphoreType.DMA,
    ],
    compiler_params=pltpu.CompilerParams(use_tc_tiling_on_sc=False,
                                         needs_layout_passes=False),
)(_embed_body)


def kernel(idx, weight):
    idx5 = (idx.T.astype(jnp.int32)
            .reshape(NT // 8, 8, NB // ROW, ROW)
            .transpose(0, 2, 1, 3)
            .reshape(NUNITS, ROW))
    out5 = _embed(idx5, weight)
    return (out5.transpose(2, 4, 0, 1, 3)
            .reshape(NB, NT, D))
